# baseline (speedup 1.0000x reference)
import functools
import math

import jax
import jax.numpy as jnp
import numpy as np
from jax.experimental import pallas as pl
from jax.experimental.pallas import tpu as pltpu

HID = 768
INTER = 1024
HEADS = 12
DH = 64
FACTOR = 5

_BF = jnp.bfloat16
_F32 = jnp.float32


def _sample_consts():
    cpu = jax.local_devices(backend="cpu")[0]
    with jax.default_device(cpu):
        key = jax.random.key(42)
        k0, k1 = jax.random.split(key)
        out = []
        for k, L in ((k0, 2048), (k1, 1024)):
            sample_k = min(FACTOR * int(math.ceil(math.log(L))), L)
            u = min(FACTOR * int(math.ceil(math.log(L))), L)
            idx = np.asarray(jax.random.randint(k, (L, sample_k), 0, L))
            cnt = np.zeros((L, L), np.float32)
            np.add.at(cnt, (np.arange(L)[:, None], idx), 1.0)
            out.append((np.asarray(jnp.asarray(cnt, _BF)), u))
    return out


(_CNT0, _U0), (_CNT1, _U1) = _sample_consts()


def _bf(a):
    return a.astype(_BF)


def _ln(y, g, b):
    mu = jnp.mean(y, axis=1, keepdims=True)
    var = jnp.mean((y - mu) ** 2, axis=1, keepdims=True)
    return (y - mu) * jax.lax.rsqrt(var + 1e-5) * g + b


def _layer_kernel(x_ref, cnt_ref, hm_ref, wq_ref, wk_ref, wv_ref, wot_ref,
                  bq_ref, bk_ref, bv_ref, bo_ref, g1_ref, be1_ref,
                  w1t_ref, b1_ref, w2t_ref, b2_ref, g2_ref, be2_ref,
                  gn_ref, bn_ref, o_ref,
                  qs, kts, vs, ms, tops, topflat, us, hs,
                  *, L, U, UP, BQ, final):
    nqb = L // BQ
    scale = 1.0 / math.sqrt(DH)

    xb = _bf(x_ref[...])
    qs[...] = _bf(
        jax.lax.dot_general(xb, wq_ref[...], (((1,), (1,)), ((), ())),
                            preferred_element_type=_F32) + bq_ref[...])
    kb = _bf(
        jax.lax.dot_general(xb, wk_ref[...], (((1,), (1,)), ((), ())),
                            preferred_element_type=_F32) + bk_ref[...])
    kts[...] = kb.T
    vs[...] = _bf(
        jax.lax.dot_general(xb, wv_ref[...], (((1,), (1,)), ((), ())),
                            preferred_element_type=_F32) + bv_ref[...])

    for qb in range(nqb):
        c = cnt_ref[qb * BQ:(qb + 1) * BQ, :]
        negm = (jnp.minimum(c.astype(_F32), 1.0) - 1.0) * 1e30
        cntk = jax.lax.dot_general(c, kts[...], (((1,), (1,)), ((), ())),
                                   preferred_element_type=_F32)
        for h in range(HEADS):
            kt_h = kts[h * DH:(h + 1) * DH, :]
            qv = qs[qb * BQ:(qb + 1) * BQ, h * DH:(h + 1) * DH]
            s = jnp.dot(qv, kt_h, preferred_element_type=_F32)
            mx = jnp.max(s + negm, axis=1, keepdims=True)
            sm = jnp.sum(qv.astype(_F32) * cntk[:, h * DH:(h + 1) * DH],
                         axis=1, keepdims=True)
            ms[qb * BQ:(qb + 1) * BQ, h:h + 1] = mx - sm * (1.0 / L)

    m = ms[...].T
    iota = jax.lax.broadcasted_iota(jnp.int32, (HEADS, L), 1)
    tops[...] = jnp.full((HEADS, UP), -1, jnp.int32)
    for u in range(U):
        mxv = jnp.max(m, axis=1, keepdims=True)
        amax = jnp.min(jnp.where(m == mxv, iota, L), axis=1, keepdims=True)
        tops[:, u:u + 1] = amax
        m = jnp.where(iota == amax, -jnp.inf, m)
    for h in range(HEADS):
        topflat[0:1, h * UP:(h + 1) * UP] = tops[h:h + 1, :]

    ptt = (jax.lax.broadcasted_iota(jnp.int32, (L, HEADS * UP), 0)
           == topflat[...]).astype(_BF)
    qr = jax.lax.dot_general(ptt, qs[...], (((0,), (0,)), ((), ())),
                             preferred_element_type=_F32)
    hmf = hm_ref[...].astype(_F32)
    qrp = _bf(qr * hmf)
    R = HEADS * UP
    RB = R // 2
    for rb in range(0, R, RB):
        sc = jax.lax.dot_general(qrp[rb:rb + RB], kts[...],
                                 (((1,), (0,)), ((), ())),
                                 preferred_element_type=_F32) * scale
        sc = sc - jnp.max(sc, axis=1, keepdims=True)
        e = jnp.exp(sc)
        attn = e * (1.0 / jnp.sum(e, axis=1, keepdims=True))
        us[rb:rb + RB, :] = jnp.dot(_bf(attn), vs[...],
                                    preferred_element_type=_F32)
    mv = jnp.mean(vs[...].astype(_F32), axis=0, keepdims=True)
    d_all = jnp.dot(_bf((us[...] - mv) * hmf), wot_ref[...],
                    preferred_element_type=_F32)
    rv = bo_ref[...] + jnp.dot(_bf(mv), wot_ref[...],
                               preferred_element_type=_F32)
    o = (x_ref[...] + rv
         + jnp.dot(ptt, _bf(d_all), preferred_element_type=_F32))
    o = _ln(o, g1_ref[...], be1_ref[...])

    hs[...] = _bf(jnp.maximum(
        jnp.dot(_bf(o), w1t_ref[...], preferred_element_type=_F32)
        + b1_ref[...], 0.0))
    y = o + jnp.dot(hs[...], w2t_ref[...], preferred_element_type=_F32) \
        + b2_ref[...]
    y = _ln(y, g2_ref[...], be2_ref[...])
    if final:
        y = _ln(y, gn_ref[...], bn_ref[...])
    o_ref[...] = y


def _head_mask(UP):
    hm = np.zeros((HEADS * UP, HID), np.float32)
    for h in range(HEADS):
        hm[h * UP:(h + 1) * UP, h * DH:(h + 1) * DH] = 1.0
    return hm


def _attn_ffn_layer(x2, p, cnt, U, final, gn, bn):
    L = x2.shape[0]
    UP = (U + 7) // 8 * 8
    BQ = 512
    row = lambda a: a.reshape(1, -1)
    return pl.pallas_call(
        functools.partial(_layer_kernel, L=L, U=U, UP=UP, BQ=BQ,
                          final=final),
        out_shape=jax.ShapeDtypeStruct((L, HID), _F32),
        scratch_shapes=[
            pltpu.VMEM((L, HID), _BF),
            pltpu.VMEM((HID, L), _BF),
            pltpu.VMEM((L, HID), _BF),
            pltpu.VMEM((L, HEADS), _F32),
            pltpu.VMEM((HEADS, UP), jnp.int32),
            pltpu.VMEM((1, HEADS * UP), jnp.int32),
            pltpu.VMEM((HEADS * UP, HID), _F32),
            pltpu.VMEM((L, INTER), _BF),
        ],
    )(x2, cnt, jnp.asarray(_head_mask(UP), _BF),
      _bf(p["Wq"]), _bf(p["Wk"]), _bf(p["Wv"]), _bf(p["Wo"].T),
      row(p["bq"]), row(p["bk"]), row(p["bv"]), row(p["bo"]),
      row(p["g1"]), row(p["be1"]),
      _bf(p["W1"].T), row(p["b1"]), _bf(p["W2"].T), row(p["b2"]),
      row(p["g2"]), row(p["be2"]), row(gn), row(bn))


def _distill_kernel(xp_ref, w_ref, cb_ref, bng_ref, bnb_ref, o_ref, *, L):
    xp = _bf(xp_ref[...])
    dn = (((1,), (1,)), ((), ()))
    y = (jax.lax.dot_general(xp[0:L, :], w_ref[0], dn,
                             preferred_element_type=_F32)
         + jax.lax.dot_general(xp[1:L + 1, :], w_ref[1], dn,
                               preferred_element_type=_F32)
         + jax.lax.dot_general(xp[2:L + 2, :], w_ref[2], dn,
                               preferred_element_type=_F32)
         + cb_ref[...])
    mu = jnp.mean(y, axis=0, keepdims=True)
    var = jnp.mean((y - mu) ** 2, axis=0, keepdims=True)
    y = (y - mu) * jax.lax.rsqrt(var + 1e-5) * bng_ref[...] + bnb_ref[...]
    y = jnp.where(y > 0.0, y, jnp.exp(y) - 1.0)
    ninf = jnp.full((1, y.shape[1]), -jnp.inf, jnp.float32)
    ym1 = jnp.concatenate([ninf, y[:L - 1]], axis=0)
    yp1 = jnp.concatenate([y[1:], ninf], axis=0)
    o_ref[...] = jnp.maximum(jnp.maximum(ym1, y), yp1)


def _distill(x2, p):
    L = x2.shape[0]
    xp = jnp.concatenate([x2[-1:], x2, x2[:1]], axis=0)
    wT = _bf(jnp.transpose(p["convW"], (2, 0, 1)))
    b = pl.pallas_call(
        functools.partial(_distill_kernel, L=L),
        out_shape=jax.ShapeDtypeStruct((L, HID), _F32),
    )(xp, wT, p["convb"].reshape(1, HID), p["bng"].reshape(1, HID),
      p["bnb"].reshape(1, HID))
    return b[::2]


def kernel(x, params):
    x2 = x[0]
    x2 = _attn_ffn_layer(x2, params["layer0"], _CNT0, _U0, False,
                         params["gN"], params["bN"])
    x2 = _distill(x2, params["distill"])
    x2 = _attn_ffn_layer(x2, params["layer1"], _CNT1, _U1, True,
                         params["gN"], params["bN"])
    return x2[None]
